# baseline (speedup 1.0000x reference)
import functools

import jax
import jax.numpy as jnp
from jax import lax
from jax.experimental import pallas as pl
from jax.experimental.pallas import tpu as pltpu
from jax.experimental.pallas import tpu_sc as plsc

_SMOOTH = 0.1
_CONF = 1.0 - _SMOOTH
_IGN = 0


def _sc_gather_flat(predT_flat, target, N):
    info = plsc.get_sparse_core_info()
    nw = info.num_cores * info.num_subcores
    bpw = N // nw
    mesh = plsc.VectorSubcoreMesh(core_axis_name="c", subcore_axis_name="s")

    @functools.partial(
        pl.kernel, mesh=mesh,
        out_type=jax.ShapeDtypeStruct((N,), jnp.float32),
        scratch_types=[
            pltpu.VMEM((bpw,), jnp.int32),
            pltpu.VMEM((bpw,), jnp.int32),
            pltpu.VMEM((bpw,), jnp.float32),
            pltpu.SemaphoreType.DMA,
        ],
    )
    def sc_gather(pred_hbm, tgt_hbm, out_hbm, tgt_v, idx_v, val_v, sem):
        wid = lax.axis_index("s") * info.num_cores + lax.axis_index("c")
        base = wid * bpw
        pltpu.sync_copy(tgt_hbm.at[pl.ds(base, bpw)], tgt_v)
        for g in range(bpw // 16):
            lanes = base + g * 16 + lax.broadcasted_iota(jnp.int32, (16,), 0)
            idx_v[pl.ds(g * 16, 16)] = (
                tgt_v[pl.ds(g * 16, 16)] * N + lanes)
        pltpu.async_copy(pred_hbm.at[idx_v], val_v, sem).wait()
        pltpu.sync_copy(val_v, out_hbm.at[pl.ds(base, bpw)])

    return sc_gather(predT_flat, target)


def _tc_body(predT_ref, tgt_ref, predt_ref, out_ref, macc, sacc, spacc, p0,
             *, C, N, H):
    j = pl.program_id(0)
    nj = pl.num_programs(0)
    G = H // 8
    sub8c = lax.broadcasted_iota(jnp.int32, (8, 128), 0)

    @pl.when(j == 0)
    def _init():
        macc[...] = jnp.full((8, N), -jnp.inf, jnp.float32)
        sacc[...] = jnp.zeros((8, N), jnp.float32)
        spacc[...] = jnp.zeros((8, N), jnp.float32)
        p0[...] = predT_ref[0:1, :]

    t = tgt_ref[...]

    def process(ng, rem):
        ngt = ng + (1 if rem else 0)
        for c in range(N // 128):
            cs = slice(c * 128, (c + 1) * 128)

            def load(g):
                x = predT_ref[g * 8:(g + 1) * 8, cs]
                if rem and g == ng:
                    x = jnp.where(sub8c < rem, x, -jnp.inf)
                return x

            bm = load(0)
            for g in range(1, ngt):
                bm = jnp.maximum(bm, load(g))
            m_old = macc[:, cs]
            mnew = jnp.maximum(m_old, bm)
            scale = jnp.exp(m_old - mnew)
            macc[:, cs] = mnew

            se0 = jnp.zeros((8, 128), jnp.float32)
            se1 = jnp.zeros((8, 128), jnp.float32)
            sp = jnp.zeros((8, 128), jnp.float32)
            for g in range(ngt):
                x = load(g)
                e = jnp.exp(x - mnew)
                if g % 2 == 0:
                    se0 = se0 + e
                else:
                    se1 = se1 + e
                if rem and g == ng:
                    x = jnp.where(sub8c < rem, x, 0.0)
                sp = sp + x
            sacc[:, cs] = sacc[:, cs] * scale + (se0 + se1)
            spacc[:, cs] = spacc[:, cs] + sp

    tail = C - (nj - 1) * H

    @pl.when(j < nj - 1)
    def _main():
        process(G, 0)

    @pl.when(j == nj - 1)
    def _fin():
        process(tail // 8, tail % 8)
        M = jnp.max(macc[...], axis=0, keepdims=True)
        S = jnp.sum(sacc[...] * jnp.exp(macc[...] - M), axis=0,
                    keepdims=True)
        SP = jnp.sum(spacc[...], axis=0, keepdims=True)
        PT = predt_ref[...]
        lse = M + jnp.log(S)
        logpt = PT - lse
        logp0 = p0[...] - lse
        ssum = SP - C * lse
        row = (_SMOOTH / (C - 1)) * (ssum - logp0 - logpt) + _CONF * logpt
        row = jnp.where(t == _IGN, 0.0, -row)
        out_ref[0, 0] = jnp.sum(row) / N


def kernel(pred, target):
    N, C = pred.shape
    H = 2048
    nj = -(-C // H)
    predT = pred.T
    t32 = target.astype(jnp.int32)
    t2 = t32.reshape(1, N)
    predt = _sc_gather_flat(predT.reshape(-1), t32, N).reshape(1, N)
    out = pl.pallas_call(
        functools.partial(_tc_body, C=C, N=N, H=H),
        grid=(nj,),
        in_specs=[
            pl.BlockSpec((H, N), lambda j: (j, 0)),
            pl.BlockSpec((1, N), lambda j: (0, 0)),
            pl.BlockSpec((1, N), lambda j: (0, 0)),
        ],
        out_specs=pl.BlockSpec((1, 1), lambda j: (0, 0),
                               memory_space=pltpu.SMEM),
        out_shape=jax.ShapeDtypeStruct((1, 1), jnp.float32),
        scratch_shapes=[
            pltpu.VMEM((8, N), jnp.float32),
            pltpu.VMEM((8, N), jnp.float32),
            pltpu.VMEM((8, N), jnp.float32),
            pltpu.VMEM((1, N), jnp.float32),
        ],
        compiler_params=pltpu.CompilerParams(
            dimension_semantics=("arbitrary",)),
    )(predT, t2, predt)
    return out[0, 0]


# speedup vs baseline: 2.8725x; 2.8725x over previous
import functools

import jax
import jax.numpy as jnp
from jax import lax
from jax.experimental import pallas as pl
from jax.experimental.pallas import tpu as pltpu

_SMOOTH = 0.1
_CONF = 1.0 - _SMOOTH
_IGN = 0


def _tc_body(predT_ref, tgt_ref, out_ref, macc, sacc, spacc, ptacc, p0,
             *, C, N, H):
    j = pl.program_id(0)
    nj = pl.num_programs(0)
    G = H // 8
    sub8 = lax.broadcasted_iota(jnp.int32, (8, N), 0)

    @pl.when(j == 0)
    def _init():
        macc[...] = jnp.full((8, N), -jnp.inf, jnp.float32)
        sacc[...] = jnp.zeros((8, N), jnp.float32)
        spacc[...] = jnp.zeros((8, N), jnp.float32)
        ptacc[...] = jnp.zeros((8, N), jnp.float32)
        p0[...] = predT_ref[0:1, :]

    t = tgt_ref[...]
    sub8c = lax.broadcasted_iota(jnp.int32, (8, 128), 0)

    def process(ng, rem):
        ngt = ng + (1 if rem else 0)
        for c in range(N // 128):
            cs = slice(c * 128, (c + 1) * 128)

            def load(g):
                x = predT_ref[g * 8:(g + 1) * 8, cs]
                if rem and g == ng:
                    x = jnp.where(sub8c < rem, x, -jnp.inf)
                return x

            bm = load(0)
            for g in range(1, ngt):
                bm = jnp.maximum(bm, load(g))
            m_old = macc[:, cs]
            mnew = jnp.maximum(m_old, bm)
            scale = jnp.exp(m_old - mnew)
            macc[:, cs] = mnew

            tjs = (jnp.broadcast_to(t[:, cs], (8, 128))
                   - (j * H) - sub8c)
            se0 = jnp.zeros((8, 128), jnp.float32)
            se1 = jnp.zeros((8, 128), jnp.float32)
            sp = jnp.zeros((8, 128), jnp.float32)
            pt = jnp.zeros((8, 128), jnp.float32)
            for g in range(ngt):
                x = load(g)
                e = jnp.exp(x - mnew)
                if g % 2 == 0:
                    se0 = se0 + e
                else:
                    se1 = se1 + e
                if rem and g == ng:
                    x = jnp.where(sub8c < rem, x, 0.0)
                sp = sp + x
                pt = pt + jnp.where(tjs == g * 8, x, 0.0)
            sacc[:, cs] = sacc[:, cs] * scale + (se0 + se1)
            spacc[:, cs] = spacc[:, cs] + sp
            ptacc[:, cs] = ptacc[:, cs] + pt

    tail = C - (nj - 1) * H

    @pl.when(j < nj - 1)
    def _main():
        process(G, 0)

    @pl.when(j == nj - 1)
    def _fin():
        process(tail // 8, tail % 8)
        M = jnp.max(macc[...], axis=0, keepdims=True)
        S = jnp.sum(sacc[...] * jnp.exp(macc[...] - M), axis=0,
                    keepdims=True)
        SP = jnp.sum(spacc[...], axis=0, keepdims=True)
        PT = jnp.sum(ptacc[...], axis=0, keepdims=True)
        lse = M + jnp.log(S)
        logpt = PT - lse
        logp0 = p0[...] - lse
        ssum = SP - C * lse
        row = (_SMOOTH / (C - 1)) * (ssum - logp0 - logpt) + _CONF * logpt
        row = jnp.where(t == _IGN, 0.0, -row)
        out_ref[0, 0] = jnp.sum(row) / N


def kernel(pred, target):
    N, C = pred.shape
    H = 2048
    nj = -(-C // H)
    predT = pred.T
    t2 = target.astype(jnp.int32).reshape(1, N)
    out = pl.pallas_call(
        functools.partial(_tc_body, C=C, N=N, H=H),
        grid=(nj,),
        in_specs=[
            pl.BlockSpec((H, N), lambda j: (j, 0)),
            pl.BlockSpec((1, N), lambda j: (0, 0)),
        ],
        out_specs=pl.BlockSpec((1, 1), lambda j: (0, 0),
                               memory_space=pltpu.SMEM),
        out_shape=jax.ShapeDtypeStruct((1, 1), jnp.float32),
        scratch_shapes=[
            pltpu.VMEM((8, N), jnp.float32),
            pltpu.VMEM((8, N), jnp.float32),
            pltpu.VMEM((8, N), jnp.float32),
            pltpu.VMEM((8, N), jnp.float32),
            pltpu.VMEM((1, N), jnp.float32),
        ],
        compiler_params=pltpu.CompilerParams(
            dimension_semantics=("arbitrary",)),
    )(predT, t2)
    return out[0, 0]
